# baseline (speedup 1.0000x reference)
import functools

import numpy as np
import jax
import jax.numpy as jnp
from jax import lax
from jax.experimental import pallas as pl
from jax.experimental.pallas import tpu as pltpu
from jax.experimental.pallas import tpu_sc as plsc

UNK_ID = 1
UNK_RATE = 0.5
PRIVACY_POS_IDS = (5, 7, 12, 18, 23)
B, L, C = 4096, 200, 16
VOCAB = 100000
N = B * L
NC, NS = 2, 16
NW = NC * NS
CHUNK = 1024
NCHUNK = (N // CHUNK) // NW
QPL = B // CHUNK
NVEC = CHUNK // 16


@functools.lru_cache(maxsize=1)
def _unk_i32() -> np.ndarray:
    k1, k2 = np.uint32(0), np.uint32(42)
    x0 = np.zeros(N, np.uint32)
    x1 = np.arange(N, dtype=np.uint32)
    rotations = (np.array([13, 15, 26, 6], np.uint32),
                 np.array([17, 29, 16, 24], np.uint32))
    ks = (k1, k2, k1 ^ k2 ^ np.uint32(0x1BD11BDA))
    x0 = x0 + ks[0]
    x1 = x1 + ks[1]
    for i in range(5):
        for rot in rotations[i % 2]:
            x0 = x0 + x1
            x1 = (x1 << rot) | (x1 >> (np.uint32(32) - rot))
            x1 = x0 ^ x1
        x0 = x0 + ks[(i + 1) % 3]
        x1 = x1 + ks[(i + 2) % 3] + np.uint32(i + 1)
    bits = x0 ^ x1
    f = ((bits >> np.uint32(9)) | np.uint32(0x3F800000)).view(np.float32)
    u = np.maximum(np.float32(0.0), f - np.float32(1.0))
    return (u < UNK_RATE).astype(np.int32).reshape(B, L).T.ravel()


def _body(word_h, pos_h, mask_h, unk_h, lut_h,
          obfw_h, pri_h, obfm_h, cpy_h, chars_h,
          word_v0, word_v1, pos_v0, pos_v1, mask_v0, mask_v1, unk_v0, unk_v1,
          obfw_v0, obfw_v1, pri_v0, pri_v1, obfm_v0, obfm_v1,
          cpy_v0, cpy_v1, gidx_v0, gidx_v1, rows_v0, rows_v1,
          rowst_v0, rowst_v1,
          isem0, isem1, gsem0, gsem1, osem0, osem1):
    sid = lax.axis_index("s")
    wid = sid * NC + lax.axis_index("c")

    word_v = (word_v0, word_v1)
    pos_v = (pos_v0, pos_v1)
    mask_v = (mask_v0, mask_v1)
    unk_v = (unk_v0, unk_v1)
    obfw_v = (obfw_v0, obfw_v1)
    pri_v = (pri_v0, pri_v1)
    obfm_v = (obfm_v0, obfm_v1)
    cpy_v = (cpy_v0, cpy_v1)
    gidx_v = (gidx_v0, gidx_v1)
    rows_v = (rows_v0, rows_v1)
    rowst_v = (rowst_v0, rowst_v1)
    isem = (isem0, isem1)
    gsem = (gsem0, gsem1)
    osem = (osem0, osem1)

    iota16 = jnp.arange(16, dtype=jnp.int32)

    def cbase(j):
        return (wid * NCHUNK + j) * CHUNK

    in_d, g_d, out_d = {}, {}, {}

    def fire_in(j):
        b, base = j % 2, cbase(j)
        in_d[j] = [
            pltpu.async_copy(word_h.at[pl.ds(base, CHUNK)], word_v[b], isem[b]),
            pltpu.async_copy(pos_h.at[pl.ds(base, CHUNK)], pos_v[b], isem[b]),
            pltpu.async_copy(mask_h.at[pl.ds(base, CHUNK)], mask_v[b], isem[b]),
            pltpu.async_copy(unk_h.at[pl.ds(base, CHUNK)], unk_v[b], isem[b]),
        ]

    def compute(j):
        b = j % 2

        def vec(i, carry):
            sl = pl.ds(i * 16, 16)
            w = word_v[b][sl]
            p = pos_v[b][sl]
            m = mask_v[b][sl]
            u = unk_v[b][sl]
            pri = (p == 5) | (p == 7) | (p == 12) | (p == 18) | (p == 23)
            obf = pri & (u != 0)
            cp = (m != 0) ^ obf
            i1, i0 = jnp.int32(1), jnp.int32(0)
            ow = jnp.where(obf, jnp.int32(UNK_ID), w)
            obfw_v[b][sl] = ow
            gidx_v[b][sl] = ow << 3
            pri_v[b][sl] = jnp.where(pri, i1, i0)
            obfm_v[b][sl] = jnp.where(obf, i1, i0)
            cpy_v[b][sl] = jnp.where(cp, i1, i0)
            return carry

        lax.fori_loop(0, NVEC, vec, 0)

    def fire_gather(j):
        b = j % 2
        g_d[j] = pltpu.async_copy(lut_h.at[gidx_v[b]], rows_v[b], gsem[b])

    def transpose(j):
        b = j % 2

        def tt(i, carry):
            t = i // 8
            c8 = i % 8
            for g in range(2):
                col_idx = jnp.full((16,), g * 8, jnp.int32) + c8
                for s in range(8):
                    row_idx = t * 128 + s * 16 + iota16
                    v = plsc.load_gather(rows_v[b], [row_idx, col_idx])
                    rowst_v[b][g, t, c8, pl.ds(s * 16, 16)] = v
            return carry

        lax.fori_loop(0, 64, tt, 0)

    def fire_out(j):
        b, base = j % 2, cbase(j)
        k = wid * NCHUNK + j
        l = k // QPL
        out_d[j] = [
            pltpu.async_copy(obfw_v[b], obfw_h.at[pl.ds(base, CHUNK)], osem[b]),
            pltpu.async_copy(pri_v[b], pri_h.at[pl.ds(base, CHUNK)], osem[b]),
            pltpu.async_copy(obfm_v[b], obfm_h.at[pl.ds(base, CHUNK)], osem[b]),
            pltpu.async_copy(cpy_v[b], cpy_h.at[pl.ds(base, CHUNK)], osem[b]),
            pltpu.async_copy(rowst_v[b].at[0],
                             chars_h.at[l, 0, pl.ds((k % QPL) * 8, 8)],
                             osem[b]),
            pltpu.async_copy(rowst_v[b].at[1],
                             chars_h.at[l, 1, pl.ds((k % QPL) * 8, 8)],
                             osem[b]),
        ]

    fire_in(0)
    for j in range(NCHUNK):
        for d in in_d.pop(j):
            d.wait()
        if j >= 2:
            for d in out_d.pop(j - 2):
                d.wait()
        compute(j)
        fire_gather(j)
        if j + 1 < NCHUNK:
            fire_in(j + 1)
        if j >= 1:
            g_d.pop(j - 1).wait()
            transpose(j - 1)
            fire_out(j - 1)
    g_d.pop(NCHUNK - 1).wait()
    transpose(NCHUNK - 1)
    fire_out(NCHUNK - 1)
    for j in (NCHUNK - 2, NCHUNK - 1):
        for d in out_d.pop(j):
            d.wait()


def kernel(inp_word, inp_char, inp_pos, inp_mask, lut):
    word = inp_word.T.reshape(N)
    pos = inp_pos.T.reshape(N)
    msk = inp_mask.T.reshape(N)
    unk = jnp.asarray(_unk_i32())
    lutp = (jnp.zeros((VOCAB, 128), jnp.int32).at[:, :C].set(lut)
            .reshape(VOCAB * 8, C))

    out_type = (
        jax.ShapeDtypeStruct((N,), jnp.int32),
        jax.ShapeDtypeStruct((N,), jnp.int32),
        jax.ShapeDtypeStruct((N,), jnp.int32),
        jax.ShapeDtypeStruct((N,), jnp.int32),
        jax.ShapeDtypeStruct((L, 2, B // 128, 8, 128), jnp.int32),
    )
    fn = pl.kernel(
        _body,
        out_type=out_type,
        mesh=plsc.VectorSubcoreMesh(core_axis_name="c", subcore_axis_name="s",
                                    num_cores=NC, num_subcores=NS),
        compiler_params=pltpu.CompilerParams(use_tc_tiling_on_sc=False,
                                             needs_layout_passes=False),
        scratch_types=(
            [pltpu.VMEM((CHUNK,), jnp.int32)] * 18
            + [pltpu.VMEM((CHUNK, C), jnp.int32)] * 2
            + [pltpu.VMEM((2, 8, 8, 128), jnp.int32)] * 2
            + [pltpu.SemaphoreType.DMA] * 6
        ),
    )
    obfw, pri, obfm, cpy, chars = fn(word, pos, msk, unk, lutp)

    obf_word = obfw.reshape(L, B).T
    pri_mask = pri.reshape(L, B).T.astype(bool)
    obf_mask = obfm.reshape(L, B).T.astype(bool)
    cpy_mask = cpy.reshape(L, B).T.astype(bool)
    obf_char = (chars.transpose(2, 4, 0, 1, 3)
                .reshape(B, L, C).astype(inp_char.dtype))
    return obf_word, pri_mask, obf_mask, inp_pos, obf_char, cpy_mask


# speedup vs baseline: 1.1790x; 1.1790x over previous
import functools

import numpy as np
import jax
import jax.numpy as jnp
from jax import lax
from jax.experimental import pallas as pl
from jax.experimental.pallas import tpu as pltpu
from jax.experimental.pallas import tpu_sc as plsc

UNK_ID = 1
UNK_RATE = 0.5
PRIVACY_POS_IDS = (5, 7, 12, 18, 23)
B, L, C = 4096, 200, 16
VOCAB = 100000
N = B * L
NC, NS = 2, 16
NW = NC * NS
PER_W = N // NW

ACHUNK = 3200
NACH = PER_W // ACHUNK
BCHUNK = 1024
NBCH = PER_W // BCHUNK
QPL = B // BCHUNK


@functools.lru_cache(maxsize=1)
def _unk_i32() -> np.ndarray:
    k1, k2 = np.uint32(0), np.uint32(42)
    x0 = np.zeros(N, np.uint32)
    x1 = np.arange(N, dtype=np.uint32)
    rotations = (np.array([13, 15, 26, 6], np.uint32),
                 np.array([17, 29, 16, 24], np.uint32))
    ks = (k1, k2, k1 ^ k2 ^ np.uint32(0x1BD11BDA))
    x0 = x0 + ks[0]
    x1 = x1 + ks[1]
    for i in range(5):
        for rot in rotations[i % 2]:
            x0 = x0 + x1
            x1 = (x1 << rot) | (x1 >> (np.uint32(32) - rot))
            x1 = x0 ^ x1
        x0 = x0 + ks[(i + 1) % 3]
        x1 = x1 + ks[(i + 2) % 3] + np.uint32(i + 1)
    bits = x0 ^ x1
    f = ((bits >> np.uint32(9)) | np.uint32(0x3F800000)).view(np.float32)
    u = np.maximum(np.float32(0.0), f - np.float32(1.0))
    return (u < UNK_RATE).astype(np.int32).reshape(B, L).T.ravel()


def _body_a(word_h, pos_h, mask_h, unk_h,
            obfw_h, pri_h, obfm_h, cpy_h,
            word_v0, word_v1, pos_v0, pos_v1, mask_v0, mask_v1,
            unk_v0, unk_v1, obfw_v0, obfw_v1, pri_v0, pri_v1,
            obfm_v0, obfm_v1, cpy_v0, cpy_v1,
            isem0, isem1, osem0, osem1):
    wid = lax.axis_index("s") * NC + lax.axis_index("c")

    word_v = (word_v0, word_v1)
    pos_v = (pos_v0, pos_v1)
    mask_v = (mask_v0, mask_v1)
    unk_v = (unk_v0, unk_v1)
    obfw_v = (obfw_v0, obfw_v1)
    pri_v = (pri_v0, pri_v1)
    obfm_v = (obfm_v0, obfm_v1)
    cpy_v = (cpy_v0, cpy_v1)
    isem = (isem0, isem1)
    osem = (osem0, osem1)

    in_d, out_d = {}, {}

    def cbase(j):
        return wid * PER_W + j * ACHUNK

    def fire_in(j):
        b, base = j % 2, cbase(j)
        in_d[j] = [
            pltpu.async_copy(word_h.at[pl.ds(base, ACHUNK)], word_v[b], isem[b]),
            pltpu.async_copy(pos_h.at[pl.ds(base, ACHUNK)], pos_v[b], isem[b]),
            pltpu.async_copy(mask_h.at[pl.ds(base, ACHUNK)], mask_v[b], isem[b]),
            pltpu.async_copy(unk_h.at[pl.ds(base, ACHUNK)], unk_v[b], isem[b]),
        ]

    def compute(j):
        b = j % 2

        def vec(i, carry):
            sl = pl.ds(i * 16, 16)
            w = word_v[b][sl]
            p = pos_v[b][sl]
            m = mask_v[b][sl]
            u = unk_v[b][sl]
            pri = (p == 5) | (p == 7) | (p == 12) | (p == 18) | (p == 23)
            obf = pri & (u != 0)
            cp = (m != 0) ^ obf
            i1, i0 = jnp.int32(1), jnp.int32(0)
            obfw_v[b][sl] = jnp.where(obf, jnp.int32(UNK_ID), w)
            pri_v[b][sl] = jnp.where(pri, i1, i0)
            obfm_v[b][sl] = jnp.where(obf, i1, i0)
            cpy_v[b][sl] = jnp.where(cp, i1, i0)
            return carry

        lax.fori_loop(0, ACHUNK // 16, vec, 0)

    def fire_out(j):
        b, base = j % 2, cbase(j)
        out_d[j] = [
            pltpu.async_copy(obfw_v[b], obfw_h.at[pl.ds(base, ACHUNK)], osem[b]),
            pltpu.async_copy(pri_v[b], pri_h.at[pl.ds(base, ACHUNK)], osem[b]),
            pltpu.async_copy(obfm_v[b], obfm_h.at[pl.ds(base, ACHUNK)], osem[b]),
            pltpu.async_copy(cpy_v[b], cpy_h.at[pl.ds(base, ACHUNK)], osem[b]),
        ]

    fire_in(0)
    for j in range(NACH):
        for d in in_d.pop(j):
            d.wait()
        if j >= 2:
            for d in out_d.pop(j - 2):
                d.wait()
        compute(j)
        if j + 1 < NACH:
            fire_in(j + 1)
        fire_out(j)
    for j in (NACH - 2, NACH - 1):
        for d in out_d.pop(j):
            d.wait()


def _body_b(obfw_h, lut_h, chars_h,
            idx_v0, idx_v1, rows_v0, rows_v1, rowst_v0, rowst_v1,
            isem0, isem1, gsem0, gsem1, osem0, osem1):
    wid = lax.axis_index("s") * NC + lax.axis_index("c")

    idx_v = (idx_v0, idx_v1)
    rows_v = (rows_v0, rows_v1)
    rowst_v = (rowst_v0, rowst_v1)
    isem = (isem0, isem1)
    gsem = (gsem0, gsem1)
    osem = (osem0, osem1)

    iota16 = jnp.arange(16, dtype=jnp.int32)
    in_d, g_d, out_d = {}, {}, {}

    def fire_in(j):
        b = j % 2
        base = wid * PER_W + j * BCHUNK
        in_d[j] = pltpu.async_copy(obfw_h.at[pl.ds(base, BCHUNK)], idx_v[b],
                                   isem[b])

    def fire_gather(j):
        b = j % 2
        g_d[j] = pltpu.async_copy(lut_h.at[idx_v[b]], rows_v[b], gsem[b])

    def transpose(j):
        b = j % 2

        def tt(i, carry):
            t = i // 8
            c8 = i % 8
            for g in range(2):
                col_idx = jnp.full((16,), g * 8, jnp.int32) + c8
                for s in range(8):
                    row_idx = t * 128 + s * 16 + iota16
                    v = plsc.load_gather(rows_v[b], [row_idx, col_idx])
                    rowst_v[b][g, t, c8, pl.ds(s * 16, 16)] = v
            return carry

        lax.fori_loop(0, 64, tt, 0)

    def fire_out(j):
        b = j % 2
        k = wid * NBCH + j
        l = k // QPL
        q = k % QPL
        out_d[j] = [
            pltpu.async_copy(rowst_v[b].at[0],
                             chars_h.at[l, 0, pl.ds(q * 8, 8)], osem[b]),
            pltpu.async_copy(rowst_v[b].at[1],
                             chars_h.at[l, 1, pl.ds(q * 8, 8)], osem[b]),
        ]

    fire_in(0)
    fire_in(1)
    for j in range(NBCH):
        in_d.pop(j).wait()
        fire_gather(j)
        if j >= 1:
            g_d.pop(j - 1).wait()
            if j + 1 < NBCH and j + 1 > 1:
                fire_in(j + 1)
            if j >= 3:
                for d in out_d.pop(j - 3):
                    d.wait()
            transpose(j - 1)
            fire_out(j - 1)
    g_d.pop(NBCH - 1).wait()
    for d in out_d.pop(NBCH - 3):
        d.wait()
    transpose(NBCH - 1)
    fire_out(NBCH - 1)
    for j in (NBCH - 2, NBCH - 1):
        for d in out_d.pop(j):
            d.wait()


_MESH = dict(
    mesh=plsc.VectorSubcoreMesh(core_axis_name="c", subcore_axis_name="s",
                                num_cores=NC, num_subcores=NS),
    compiler_params=pltpu.CompilerParams(use_tc_tiling_on_sc=False,
                                         needs_layout_passes=False),
)


def kernel(inp_word, inp_char, inp_pos, inp_mask, lut):
    word = inp_word.T.reshape(N)
    pos = inp_pos.T.reshape(N)
    msk = inp_mask.T.reshape(N)
    unk = jnp.asarray(_unk_i32())

    fn_a = pl.kernel(
        _body_a,
        out_type=(
            jax.ShapeDtypeStruct((N,), jnp.int32),
            jax.ShapeDtypeStruct((N,), jnp.int32),
            jax.ShapeDtypeStruct((N,), jnp.int32),
            jax.ShapeDtypeStruct((N,), jnp.int32),
        ),
        scratch_types=(
            [pltpu.VMEM((ACHUNK,), jnp.int32)] * 16
            + [pltpu.SemaphoreType.DMA] * 4
        ),
        **_MESH,
    )
    obfw, pri, obfm, cpy = fn_a(word, pos, msk, unk)

    fn_b = pl.kernel(
        _body_b,
        out_type=(
            jax.ShapeDtypeStruct((L, 2, B // 128, 8, 128), jnp.int32),
        ),
        scratch_types=(
            [pltpu.VMEM((BCHUNK,), jnp.int32)] * 2
            + [pltpu.VMEM((BCHUNK, C), jnp.int32)] * 2
            + [pltpu.VMEM((2, 8, 8, 128), jnp.int32)] * 2
            + [pltpu.SemaphoreType.DMA] * 6
        ),
        **_MESH,
    )
    (chars,) = fn_b(obfw, lut)

    obf_word = obfw.reshape(L, B).T
    pri_mask = pri.reshape(L, B).T.astype(bool)
    obf_mask = obfm.reshape(L, B).T.astype(bool)
    cpy_mask = cpy.reshape(L, B).T.astype(bool)
    obf_char = (chars.transpose(2, 4, 0, 1, 3)
                .reshape(B, L, C).astype(inp_char.dtype))
    return obf_word, pri_mask, obf_mask, inp_pos, obf_char, cpy_mask


# speedup vs baseline: 1.1809x; 1.0016x over previous
import functools

import numpy as np
import jax
import jax.numpy as jnp
from jax import lax
from jax.experimental import pallas as pl
from jax.experimental.pallas import tpu as pltpu
from jax.experimental.pallas import tpu_sc as plsc

UNK_ID = 1
UNK_RATE = 0.5
PRIVACY_POS_IDS = (5, 7, 12, 18, 23)
B, L, C = 4096, 200, 16
VOCAB = 100000
N = B * L
NC, NS = 2, 16
NW = NC * NS
PER_W = N // NW

ACHUNK = 3200
NACH = PER_W // ACHUNK
BCHUNK = 1024
NBCH = PER_W // BCHUNK
QPL = B // BCHUNK


@functools.lru_cache(maxsize=1)
def _unk_i32() -> np.ndarray:
    k1, k2 = np.uint32(0), np.uint32(42)
    x0 = np.zeros(N, np.uint32)
    x1 = np.arange(N, dtype=np.uint32)
    rotations = (np.array([13, 15, 26, 6], np.uint32),
                 np.array([17, 29, 16, 24], np.uint32))
    ks = (k1, k2, k1 ^ k2 ^ np.uint32(0x1BD11BDA))
    x0 = x0 + ks[0]
    x1 = x1 + ks[1]
    for i in range(5):
        for rot in rotations[i % 2]:
            x0 = x0 + x1
            x1 = (x1 << rot) | (x1 >> (np.uint32(32) - rot))
            x1 = x0 ^ x1
        x0 = x0 + ks[(i + 1) % 3]
        x1 = x1 + ks[(i + 2) % 3] + np.uint32(i + 1)
    bits = x0 ^ x1
    f = ((bits >> np.uint32(9)) | np.uint32(0x3F800000)).view(np.float32)
    u = np.maximum(np.float32(0.0), f - np.float32(1.0))
    return (u < UNK_RATE).astype(np.int32).reshape(B, L).T.ravel()


def _body_a(word_h, pos_h, mask_h, unk_h,
            obfw_h, pri_h, obfm_h, cpy_h,
            word_v0, word_v1, pos_v0, pos_v1, mask_v0, mask_v1,
            unk_v0, unk_v1, obfw_v0, obfw_v1, pri_v0, pri_v1,
            obfm_v0, obfm_v1, cpy_v0, cpy_v1,
            isem0, isem1, osem0, osem1):
    wid = lax.axis_index("s") * NC + lax.axis_index("c")

    word_v = (word_v0, word_v1)
    pos_v = (pos_v0, pos_v1)
    mask_v = (mask_v0, mask_v1)
    unk_v = (unk_v0, unk_v1)
    obfw_v = (obfw_v0, obfw_v1)
    pri_v = (pri_v0, pri_v1)
    obfm_v = (obfm_v0, obfm_v1)
    cpy_v = (cpy_v0, cpy_v1)
    isem = (isem0, isem1)
    osem = (osem0, osem1)

    in_d, out_d = {}, {}

    def cbase(j):
        return wid * PER_W + j * ACHUNK

    def fire_in(j):
        b, base = j % 2, cbase(j)
        in_d[j] = [
            pltpu.async_copy(word_h.at[pl.ds(base, ACHUNK)], word_v[b], isem[b]),
            pltpu.async_copy(pos_h.at[pl.ds(base, ACHUNK)], pos_v[b], isem[b]),
            pltpu.async_copy(mask_h.at[pl.ds(base, ACHUNK)], mask_v[b], isem[b]),
            pltpu.async_copy(unk_h.at[pl.ds(base, ACHUNK)], unk_v[b], isem[b]),
        ]

    def compute(j):
        b = j % 2

        def vec(i, carry):
            sl = pl.ds(i * 16, 16)
            w = word_v[b][sl]
            p = pos_v[b][sl]
            m = mask_v[b][sl]
            u = unk_v[b][sl]
            pri = (p == 5) | (p == 7) | (p == 12) | (p == 18) | (p == 23)
            obf = pri & (u != 0)
            cp = (m != 0) ^ obf
            i1, i0 = jnp.int32(1), jnp.int32(0)
            obfw_v[b][sl] = jnp.where(obf, jnp.int32(UNK_ID), w)
            pri_v[b][sl] = jnp.where(pri, i1, i0)
            obfm_v[b][sl] = jnp.where(obf, i1, i0)
            cpy_v[b][sl] = jnp.where(cp, i1, i0)
            return carry

        lax.fori_loop(0, ACHUNK // 16, vec, 0)

    def fire_out(j):
        b, base = j % 2, cbase(j)
        out_d[j] = [
            pltpu.async_copy(obfw_v[b], obfw_h.at[pl.ds(base, ACHUNK)], osem[b]),
            pltpu.async_copy(pri_v[b], pri_h.at[pl.ds(base, ACHUNK)], osem[b]),
            pltpu.async_copy(obfm_v[b], obfm_h.at[pl.ds(base, ACHUNK)], osem[b]),
            pltpu.async_copy(cpy_v[b], cpy_h.at[pl.ds(base, ACHUNK)], osem[b]),
        ]

    fire_in(0)
    for j in range(NACH):
        for d in in_d.pop(j):
            d.wait()
        if j >= 2:
            for d in out_d.pop(j - 2):
                d.wait()
        compute(j)
        if j + 1 < NACH:
            fire_in(j + 1)
        fire_out(j)
    for j in (NACH - 2, NACH - 1):
        for d in out_d.pop(j):
            d.wait()


NBUF = 3


def _body_b(obfw_h, lut_h, chars_h,
            idx_v0, idx_v1, idx_v2, rows_v0, rows_v1, rows_v2,
            rowst_v0, rowst_v1, rowst_v2,
            isem0, isem1, isem2, gsem0, gsem1, gsem2,
            osem0, osem1, osem2):
    wid = lax.axis_index("s") * NC + lax.axis_index("c")

    idx_v = (idx_v0, idx_v1, idx_v2)
    rows_v = (rows_v0, rows_v1, rows_v2)
    rowst_v = (rowst_v0, rowst_v1, rowst_v2)
    isem = (isem0, isem1, isem2)
    gsem = (gsem0, gsem1, gsem2)
    osem = (osem0, osem1, osem2)

    iota16 = jnp.arange(16, dtype=jnp.int32)
    in_d, g_d, out_d = {}, {}, {}

    def fire_in(j):
        b = j % NBUF
        base = wid * PER_W + j * BCHUNK
        in_d[j] = pltpu.async_copy(obfw_h.at[pl.ds(base, BCHUNK)], idx_v[b],
                                   isem[b])

    def fire_gather(j):
        b = j % NBUF
        g_d[j] = pltpu.async_copy(lut_h.at[idx_v[b]], rows_v[b], gsem[b])

    def transpose(j):
        b = j % NBUF

        def tt(i, carry):
            t = i // 8
            c8 = i % 8
            for g in range(2):
                col_idx = jnp.full((16,), g * 8, jnp.int32) + c8
                for s in range(8):
                    row_idx = t * 128 + s * 16 + iota16
                    v = plsc.load_gather(rows_v[b], [row_idx, col_idx])
                    rowst_v[b][g, t, c8, pl.ds(s * 16, 16)] = v
            return carry

        lax.fori_loop(0, 64, tt, 0)

    def fire_out(j):
        b = j % NBUF
        k = wid * NBCH + j
        l = k // QPL
        q = k % QPL
        out_d[j] = [
            pltpu.async_copy(rowst_v[b].at[0],
                             chars_h.at[l, 0, pl.ds(q * 8, 8)], osem[b]),
            pltpu.async_copy(rowst_v[b].at[1],
                             chars_h.at[l, 1, pl.ds(q * 8, 8)], osem[b]),
        ]

    for j0 in range(NBUF):
        fire_in(j0)
    for j in range(NBCH):
        in_d.pop(j).wait()
        fire_gather(j)
        if j >= 2:
            g_d.pop(j - 2).wait()
            if j + 1 < NBCH and j + 1 >= NBUF:
                fire_in(j + 1)
            if j >= 2 + NBUF:
                for d in out_d.pop(j - 2 - NBUF):
                    d.wait()
            transpose(j - 2)
            fire_out(j - 2)
    for j in (NBCH - 2, NBCH - 1):
        g_d.pop(j).wait()
        if j >= 2 + NBUF - 2:
            for d in out_d.pop(j - NBUF):
                d.wait()
        transpose(j)
        fire_out(j)
    for j in range(NBCH - NBUF, NBCH):
        if j in out_d:
            for d in out_d.pop(j):
                d.wait()


_MESH = dict(
    mesh=plsc.VectorSubcoreMesh(core_axis_name="c", subcore_axis_name="s",
                                num_cores=NC, num_subcores=NS),
    compiler_params=pltpu.CompilerParams(use_tc_tiling_on_sc=False,
                                         needs_layout_passes=False),
)


def kernel(inp_word, inp_char, inp_pos, inp_mask, lut):
    word = inp_word.T.reshape(N)
    pos = inp_pos.T.reshape(N)
    msk = inp_mask.T.reshape(N)
    unk = jnp.asarray(_unk_i32())

    fn_a = pl.kernel(
        _body_a,
        out_type=(
            jax.ShapeDtypeStruct((N,), jnp.int32),
            jax.ShapeDtypeStruct((N,), jnp.int32),
            jax.ShapeDtypeStruct((N,), jnp.int32),
            jax.ShapeDtypeStruct((N,), jnp.int32),
        ),
        scratch_types=(
            [pltpu.VMEM((ACHUNK,), jnp.int32)] * 16
            + [pltpu.SemaphoreType.DMA] * 4
        ),
        **_MESH,
    )
    obfw, pri, obfm, cpy = fn_a(word, pos, msk, unk)

    fn_b = pl.kernel(
        _body_b,
        out_type=(
            jax.ShapeDtypeStruct((L, 2, B // 128, 8, 128), jnp.int32),
        ),
        scratch_types=(
            [pltpu.VMEM((BCHUNK,), jnp.int32)] * NBUF
            + [pltpu.VMEM((BCHUNK, C), jnp.int32)] * NBUF
            + [pltpu.VMEM((2, 8, 8, 128), jnp.int32)] * NBUF
            + [pltpu.SemaphoreType.DMA] * (3 * NBUF)
        ),
        **_MESH,
    )
    (chars,) = fn_b(obfw, lut)

    obf_word = obfw.reshape(L, B).T
    pri_mask = pri.reshape(L, B).T.astype(bool)
    obf_mask = obfm.reshape(L, B).T.astype(bool)
    cpy_mask = cpy.reshape(L, B).T.astype(bool)
    obf_char = (chars.transpose(2, 4, 0, 1, 3)
                .reshape(B, L, C).astype(inp_char.dtype))
    return obf_word, pri_mask, obf_mask, inp_pos, obf_char, cpy_mask
